# Initial kernel scaffold; baseline (speedup 1.0000x reference)
import jax
import jax.numpy as jnp
from jax.experimental import pallas as pl


def kernel(xyz, params):
    raise NotImplementedError("write your pallas kernel here")


# speedup vs baseline: 4.0474x; 4.0474x over previous
import functools

import jax
import jax.numpy as jnp
import numpy as np
from jax import lax
from jax.experimental import pallas as pl
from jax.experimental.pallas import tpu as pltpu

B, N, NUM_CLASSES = 1, 4096, 13
BN_EPS = 1e-5
BN_DIV = np.float32(np.sqrt(1.0 + BN_EPS))
F32 = jnp.float32
BF16 = jnp.bfloat16

SA_CFGS = [
    (4096, [0.2, 0.4, 0.6], [32, 64, 64], 1, [[32, 32, 64], [64, 64, 128], [64, 96, 128]]),
    (1024, [0.4, 0.8, 1.6], [16, 32, 32], 320, [[64, 64, 128], [128, 128, 256], [128, 128, 256]]),
    (256, [0.8, 1.6, 3.2], [16, 32, 64], 640, [[128, 128, 256], [128, 128, 256], [128, 256, 256]]),
    (64, [1.6, 3.2, 6.4], [16, 32, 64], 768, [[256, 256, 512], [256, 256, 512], [256, 512, 1024]]),
]


def _bn_relu(y, b, gamma, beta):
    y = y + b
    y = y / BN_DIV * gamma + beta
    return jnp.maximum(y, 0.0)


def _fps_body(n, npoint, xr, yr, zr, idx_ref, nx_ref):
    n8 = n // 8
    X, Y, Z = xr[...], yr[...], zr[...]
    iota = (lax.broadcasted_iota(jnp.int32, (8, n8), 0) * n8
            + lax.broadcasted_iota(jnp.int32, (8, n8), 1))

    def body(t, carry):
        dist, far = carry
        oh = iota == far
        cx = jnp.sum(jnp.where(oh, X, 0.0))
        cy = jnp.sum(jnp.where(oh, Y, 0.0))
        cz = jnp.sum(jnp.where(oh, Z, 0.0))
        idx_ref[pl.ds(t, 1), :] = jnp.full((1, 1), far, jnp.int32)
        nx_ref[pl.ds(t, 1), :] = jnp.concatenate(
            [cx.reshape(1, 1), cy.reshape(1, 1), cz.reshape(1, 1)], axis=1)
        dx = X - cx
        dy = Y - cy
        dz = Z - cz
        d = dx * dx + dy * dy + dz * dz
        dist = jnp.minimum(dist, d)
        dmax = jnp.max(dist)
        far2 = jnp.min(jnp.where(dist == dmax, iota, n))
        return dist, far2

    lax.fori_loop(0, npoint, body,
                  (jnp.full((8, n8), 1e10, F32), jnp.int32(0)))


def _fps(xyz, npoint):
    n = xyz.shape[0]
    xr = xyz[:, 0].reshape(8, n // 8)
    yr = xyz[:, 1].reshape(8, n // 8)
    zr = xyz[:, 2].reshape(8, n // 8)
    idx, nx = pl.pallas_call(
        functools.partial(_fps_body, n, npoint),
        out_shape=(jax.ShapeDtypeStruct((npoint, 1), jnp.int32),
                   jax.ShapeDtypeStruct((npoint, 3), F32)),
    )(xr, yr, zr)
    return idx[:, 0], nx


def _dist_body(q_ref, pt_ref, d_ref):
    q = q_ref[...]
    pt = pt_ref[...]
    sq = (q[:, 0:1] * q[:, 0:1] + q[:, 1:2] * q[:, 1:2]
          + q[:, 2:3] * q[:, 2:3])
    sp = (pt[0:1] * pt[0:1] + pt[1:2] * pt[1:2] + pt[2:3] * pt[2:3])
    m = lax.dot_general(q.astype(BF16), pt.astype(BF16),
                        (((1,), (0,)), ((), ())),
                        preferred_element_type=F32)
    d_ref[...] = (sq + sp) - 2.0 * m


def _sqdist(a, b):
    s, n = a.shape[0], b.shape[0]
    blk = min(s, 512)
    return pl.pallas_call(
        _dist_body,
        grid=(s // blk,),
        in_specs=[pl.BlockSpec((blk, 3), lambda i: (i, 0)),
                  pl.BlockSpec((3, n), lambda i: (0, 0))],
        out_specs=pl.BlockSpec((blk, n), lambda i: (i, 0)),
        out_shape=jax.ShapeDtypeStruct((s, n), F32),
    )(a, b.T)


def _sa_mlp_body(k, g_ref, *refs):
    (w1, b1, g1, be1, w2, b2, g2, be2, w3, b3, g3, be3, o_ref) = refs
    h = g_ref[...]
    y = lax.dot_general(h, w1[...], (((1,), (0,)), ((), ())),
                        preferred_element_type=F32)
    y = _bn_relu(y, b1[...], g1[...], be1[...])
    y = lax.dot_general(y.astype(BF16), w2[...], (((1,), (0,)), ((), ())),
                        preferred_element_type=F32)
    y = _bn_relu(y, b2[...], g2[...], be2[...])
    y = lax.dot_general(y.astype(BF16), w3[...], (((1,), (0,)), ((), ())),
                        preferred_element_type=F32)
    y = _bn_relu(y, b3[...], g3[...], be3[...])
    rows, c3 = y.shape
    o_ref[...] = jnp.max(y.reshape(rows // k, k, c3), axis=1)


def _sa_mlp(gp_bf, k, layers):
    rows, cin = gp_bf.shape
    s = rows // k
    c3 = layers[2]["w"].shape[0]
    rows_target = 1024 if c3 >= 512 else 4096
    blk_s = max(1, min(s, rows_target // k))
    args = [gp_bf]
    in_specs = [pl.BlockSpec((blk_s * k, cin), lambda i: (i, 0))]
    for lp in layers:
        co, ci = lp["w"].shape
        args += [lp["w"].T.astype(BF16), lp["b"].reshape(1, co),
                 lp["gamma"].reshape(1, co), lp["beta"].reshape(1, co)]
        in_specs += [pl.BlockSpec((ci, co), lambda i: (0, 0)),
                     pl.BlockSpec((1, co), lambda i: (0, 0)),
                     pl.BlockSpec((1, co), lambda i: (0, 0)),
                     pl.BlockSpec((1, co), lambda i: (0, 0))]
    return pl.pallas_call(
        functools.partial(_sa_mlp_body, k),
        grid=(s // blk_s,),
        in_specs=in_specs,
        out_specs=pl.BlockSpec((blk_s, c3), lambda i: (i, 0)),
        out_shape=jax.ShapeDtypeStruct((s, c3), F32),
    )(*args)


def _fp_body(x1_ref, x2t_ref, p1_ref, p2_ref, *refs):
    (w1a, w1b, b1, g1, be1, w2, b2, g2, be2, o_ref) = refs
    q = x1_ref[...]
    pt = x2t_ref[...]
    sq = (q[:, 0:1] * q[:, 0:1] + q[:, 1:2] * q[:, 1:2]
          + q[:, 2:3] * q[:, 2:3])
    sp = (pt[0:1] * pt[0:1] + pt[1:2] * pt[1:2] + pt[2:3] * pt[2:3])
    m = lax.dot_general(q.astype(BF16), pt.astype(BF16),
                        (((1,), (0,)), ((), ())), preferred_element_type=F32)
    d = (sq + sp) - 2.0 * m
    inf = F32(np.inf)
    t1 = jnp.min(d, axis=1, keepdims=True)
    t2 = jnp.min(jnp.where(d > t1, d, inf), axis=1, keepdims=True)
    t3 = jnp.min(jnp.where(d > t2, d, inf), axis=1, keepdims=True)
    recip = jnp.where(d <= t3, 1.0 / (d + F32(1e-8)), 0.0)
    w = recip / jnp.sum(recip, axis=1, keepdims=True)
    interp = lax.dot_general(w, p2_ref[...], (((1,), (0,)), ((), ())),
                             precision=lax.Precision.HIGHEST,
                             preferred_element_type=F32)
    y = (lax.dot_general(p1_ref[...], w1a[...], (((1,), (0,)), ((), ())),
                         preferred_element_type=F32)
         + lax.dot_general(interp.astype(BF16), w1b[...],
                           (((1,), (0,)), ((), ())),
                           preferred_element_type=F32))
    y = _bn_relu(y, b1[...], g1[...], be1[...])
    y = lax.dot_general(y.astype(BF16), w2[...], (((1,), (0,)), ((), ())),
                        preferred_element_type=F32)
    y = _bn_relu(y, b2[...], g2[...], be2[...])
    o_ref[...] = y


def _fp(xyz1, xyz2, points1, points2, layers):
    s1, s2 = xyz1.shape[0], xyz2.shape[0]
    c1, c2 = points1.shape[1], points2.shape[1]
    blk = min(s1, 512)
    l1, l2 = layers
    co1 = l1["w"].shape[0]
    co2 = l2["w"].shape[0]
    w1a = l1["w"][:, :c1].T.astype(BF16)
    w1b = l1["w"][:, c1:].T.astype(BF16)
    args = [xyz1, xyz2.T, points1.astype(BF16), points2,
            w1a, w1b, l1["b"].reshape(1, co1), l1["gamma"].reshape(1, co1),
            l1["beta"].reshape(1, co1),
            l2["w"].T.astype(BF16), l2["b"].reshape(1, co2),
            l2["gamma"].reshape(1, co2), l2["beta"].reshape(1, co2)]
    in_specs = [pl.BlockSpec((blk, 3), lambda i: (i, 0)),
                pl.BlockSpec((3, s2), lambda i: (0, 0)),
                pl.BlockSpec((blk, c1), lambda i: (i, 0)),
                pl.BlockSpec((s2, c2), lambda i: (0, 0)),
                pl.BlockSpec((c1, co1), lambda i: (0, 0)),
                pl.BlockSpec((c2, co1), lambda i: (0, 0)),
                pl.BlockSpec((1, co1), lambda i: (0, 0)),
                pl.BlockSpec((1, co1), lambda i: (0, 0)),
                pl.BlockSpec((1, co1), lambda i: (0, 0)),
                pl.BlockSpec((co1, co2), lambda i: (0, 0)),
                pl.BlockSpec((1, co2), lambda i: (0, 0)),
                pl.BlockSpec((1, co2), lambda i: (0, 0)),
                pl.BlockSpec((1, co2), lambda i: (0, 0))]
    return pl.pallas_call(
        _fp_body,
        grid=(s1 // blk,),
        in_specs=in_specs,
        out_specs=pl.BlockSpec((blk, co2), lambda i: (i, 0)),
        out_shape=jax.ShapeDtypeStruct((s1, co2), F32),
    )(*args)


def _head_body(x_ref, w1, b1, g1, be1, w2, b2, o_ref):
    y = lax.dot_general(x_ref[...], w1[...], (((1,), (0,)), ((), ())),
                        preferred_element_type=F32)
    y = _bn_relu(y, b1[...], g1[...], be1[...])
    z = lax.dot_general(y.astype(BF16), w2[...], (((1,), (0,)), ((), ())),
                        preferred_element_type=F32) + b2[...]
    o_ref[...] = 1.0 / (1.0 + jnp.exp(-z))


def _head(l0p, head):
    s = l0p.shape[0]
    blk = min(s, 1024)
    c1p = head["conv1"]
    c2p = head["conv2"]
    nc = c2p["w"].shape[0]
    args = [l0p.astype(BF16), c1p["w"].T.astype(BF16),
            c1p["b"].reshape(1, 128), c1p["gamma"].reshape(1, 128),
            c1p["beta"].reshape(1, 128),
            c2p["w"].T.astype(BF16), c2p["b"].reshape(1, nc)]
    in_specs = [pl.BlockSpec((blk, 128), lambda i: (i, 0)),
                pl.BlockSpec((128, 128), lambda i: (0, 0)),
                pl.BlockSpec((1, 128), lambda i: (0, 0)),
                pl.BlockSpec((1, 128), lambda i: (0, 0)),
                pl.BlockSpec((1, 128), lambda i: (0, 0)),
                pl.BlockSpec((128, nc), lambda i: (0, 0)),
                pl.BlockSpec((1, nc), lambda i: (0, 0))]
    return pl.pallas_call(
        _head_body,
        grid=(s // blk,),
        in_specs=in_specs,
        out_specs=pl.BlockSpec((blk, nc), lambda i: (i, 0)),
        out_shape=jax.ShapeDtypeStruct((s, nc), F32),
    )(*args)


def _ball_idx(d, radius, k):
    n = d.shape[1]
    mask = d <= np.float32(radius ** 2)
    iota = jnp.arange(n, dtype=jnp.int32)[None, :]
    gi = jnp.where(mask, iota, n)
    gi = -lax.top_k(-gi, k)[0]
    pad = jnp.where(gi[:, :1] == n, n - 1, gi[:, :1])
    return jnp.where(gi == n, pad, gi)


def _sa(xyz, points, cfg, params):
    npoint, radii, nsamples, _, _ = cfg
    _, new_xyz = _fps(xyz, npoint)
    d = _sqdist(new_xyz, xyz)
    feat = jnp.concatenate([points, xyz], axis=-1)
    cin = points.shape[-1]
    outs = []
    for radius, k, layers in zip(radii, nsamples, params):
        gi = _ball_idx(d, radius, k)
        g = feat[gi]
        gp = jnp.concatenate(
            [g[..., :cin], g[..., cin:] - new_xyz[:, None, :]], axis=-1)
        gp_bf = gp.reshape(-1, cin + 3).astype(BF16)
        outs.append(_sa_mlp(gp_bf, k, layers))
    return new_xyz, jnp.concatenate(outs, axis=-1)


def kernel(xyz, params):
    x0 = xyz[0]
    l0_xyz, l0_points = x0[:, :3], x0[:, 3:]
    l1_xyz, l1_points = _sa(l0_xyz, l0_points, SA_CFGS[0], params["sa"][0])
    l2_xyz, l2_points = _sa(l1_xyz, l1_points, SA_CFGS[1], params["sa"][1])
    l3_xyz, l3_points = _sa(l2_xyz, l2_points, SA_CFGS[2], params["sa"][2])
    l4_xyz, l4_points = _sa(l3_xyz, l3_points, SA_CFGS[3], params["sa"][3])
    l3_points = _fp(l3_xyz, l4_xyz, l3_points, l4_points, params["fp"][0])
    l2_points = _fp(l2_xyz, l3_xyz, l2_points, l3_points, params["fp"][1])
    l1_points = _fp(l1_xyz, l2_xyz, l1_points, l2_points, params["fp"][2])
    l0p = _fp(l0_xyz, l1_xyz, l0_points, l1_points, params["fp"][3])
    y = _head(l0p, params["head"])
    return y[None], l0p[None]


# speedup vs baseline: 7.7340x; 1.9109x over previous
import functools

import jax
import jax.numpy as jnp
import numpy as np
from jax import lax
from jax.experimental import pallas as pl
from jax.experimental.pallas import tpu as pltpu
from jax.experimental.pallas import tpu_sc as plsc

B, N, NUM_CLASSES = 1, 4096, 13
BN_EPS = 1e-5
BN_DIV = np.float32(np.sqrt(1.0 + BN_EPS))
F32 = jnp.float32
BF16 = jnp.bfloat16

SA_CFGS = [
    (4096, [0.2, 0.4, 0.6], [32, 64, 64], 1, [[32, 32, 64], [64, 64, 128], [64, 96, 128]]),
    (1024, [0.4, 0.8, 1.6], [16, 32, 32], 320, [[64, 64, 128], [128, 128, 256], [128, 128, 256]]),
    (256, [0.8, 1.6, 3.2], [16, 32, 64], 640, [[128, 128, 256], [128, 128, 256], [128, 256, 256]]),
    (64, [1.6, 3.2, 6.4], [16, 32, 64], 768, [[256, 256, 512], [256, 256, 512], [256, 512, 1024]]),
]


def _bn_relu(y, b, gamma, beta):
    y = y + b
    y = y / BN_DIV * gamma + beta
    return jnp.maximum(y, 0.0)


def _fps_body(n, npoint, xr, yr, zr, idx_ref, nx_ref):
    n8 = n // 8
    X, Y, Z = xr[...], yr[...], zr[...]
    iota = (lax.broadcasted_iota(jnp.int32, (8, n8), 0) * n8
            + lax.broadcasted_iota(jnp.int32, (8, n8), 1))

    def body(t, carry):
        dist, far = carry
        oh = iota == far
        cx = jnp.sum(jnp.where(oh, X, 0.0))
        cy = jnp.sum(jnp.where(oh, Y, 0.0))
        cz = jnp.sum(jnp.where(oh, Z, 0.0))
        idx_ref[pl.ds(t, 1), :] = jnp.full((1, 1), far, jnp.int32)
        nx_ref[pl.ds(t, 1), :] = jnp.concatenate(
            [cx.reshape(1, 1), cy.reshape(1, 1), cz.reshape(1, 1)], axis=1)
        dx = X - cx
        dy = Y - cy
        dz = Z - cz
        d = dx * dx + dy * dy + dz * dz
        dist = jnp.minimum(dist, d)
        dmax = jnp.max(dist)
        far2 = jnp.min(jnp.where(dist == dmax, iota, n))
        return dist, far2

    lax.fori_loop(0, npoint, body,
                  (jnp.full((8, n8), 1e10, F32), jnp.int32(0)))


def _fps(xyz, npoint):
    n = xyz.shape[0]
    xr = xyz[:, 0].reshape(8, n // 8)
    yr = xyz[:, 1].reshape(8, n // 8)
    zr = xyz[:, 2].reshape(8, n // 8)
    idx, nx = pl.pallas_call(
        functools.partial(_fps_body, n, npoint),
        out_shape=(jax.ShapeDtypeStruct((npoint, 1), jnp.int32),
                   jax.ShapeDtypeStruct((npoint, 3), F32)),
    )(xr, yr, zr)
    return idx[:, 0], nx


def _dist_body(q_ref, pt_ref, d_ref):
    q = q_ref[...]
    pt = pt_ref[...]
    sq = (q[:, 0:1] * q[:, 0:1] + q[:, 1:2] * q[:, 1:2]
          + q[:, 2:3] * q[:, 2:3])
    sp = (pt[0:1] * pt[0:1] + pt[1:2] * pt[1:2] + pt[2:3] * pt[2:3])
    m = lax.dot_general(q.astype(BF16), pt.astype(BF16),
                        (((1,), (0,)), ((), ())),
                        preferred_element_type=F32)
    d_ref[...] = (sq + sp) - 2.0 * m


def _sqdist(a, b):
    s, n = a.shape[0], b.shape[0]
    blk = min(s, 512)
    return pl.pallas_call(
        _dist_body,
        grid=(s // blk,),
        in_specs=[pl.BlockSpec((blk, 3), lambda i: (i, 0)),
                  pl.BlockSpec((3, n), lambda i: (0, 0))],
        out_specs=pl.BlockSpec((blk, n), lambda i: (i, 0)),
        out_shape=jax.ShapeDtypeStruct((s, n), F32),
    )(a, b.T)


def _distrank_body(r2s, q_ref, pt_ref, p_ref, cnt_ref):
    q = q_ref[...]
    pt = pt_ref[...]
    sq = (q[:, 0:1] * q[:, 0:1] + q[:, 1:2] * q[:, 1:2]
          + q[:, 2:3] * q[:, 2:3])
    sp = (pt[0:1] * pt[0:1] + pt[1:2] * pt[1:2] + pt[2:3] * pt[2:3])
    m = lax.dot_general(q.astype(BF16), pt.astype(BF16),
                        (((1,), (0,)), ((), ())),
                        preferred_element_type=F32)
    d = (sq + sp) - 2.0 * m
    n = d.shape[1]
    ri = lax.broadcasted_iota(jnp.int32, (128, 128), 0)
    ci = lax.broadcasted_iota(jnp.int32, (128, 128), 1)
    u = (ri < ci).astype(BF16)
    blk = d.shape[0]
    bases = [jnp.zeros((blk, 1), F32) for _ in r2s]
    for c in range(n // 128):
        dch = d[:, c * 128:(c + 1) * 128]
        pch = jnp.zeros((blk, 128), jnp.int32)
        clsch = jnp.zeros((blk, 128), jnp.int32)
        for b, r2 in enumerate(r2s):
            mc = (dch <= r2).astype(F32)
            clsch = clsch + mc.astype(jnp.int32)
            excl = bases[b] + lax.dot_general(
                mc.astype(BF16), u, (((1,), (0,)), ((), ())),
                preferred_element_type=F32)
            bases[b] = bases[b] + jnp.sum(mc, axis=1, keepdims=True)
            pch = pch | (jnp.minimum(excl, 255.0).astype(jnp.int32) << (8 * b))
        p_ref[:, c * 128:(c + 1) * 128] = pch | (clsch << 24)
    cnt_ref[...] = jnp.concatenate(bases, axis=1).astype(jnp.int32)


def _distrank(a, b, radii):
    s, n = a.shape[0], b.shape[0]
    blk = min(s, 512)
    r2s = tuple(np.float32(r ** 2) for r in radii)
    return pl.pallas_call(
        functools.partial(_distrank_body, r2s),
        grid=(s // blk,),
        in_specs=[pl.BlockSpec((blk, 3), lambda i: (i, 0)),
                  pl.BlockSpec((3, n), lambda i: (0, 0))],
        out_specs=(pl.BlockSpec((blk, n), lambda i: (i, 0)),
                   pl.BlockSpec((blk, 3), lambda i: (i, 0))),
        out_shape=(jax.ShapeDtypeStruct((s, n), jnp.int32),
                   jax.ShapeDtypeStruct((s, 3), jnp.int32)),
    )(a, b.T)


_SC_NC = 2
_SC_NW = 32


def _sc_compact(packed, ks):
    s, n = packed.shape
    k1, k2, k3 = ks
    rpw = s // _SC_NW
    nch = n // 16
    mesh = plsc.VectorSubcoreMesh(core_axis_name="c", subcore_axis_name="s")

    @functools.partial(
        pl.kernel, mesh=mesh,
        compiler_params=pltpu.CompilerParams(needs_layout_passes=False),
        out_type=(jax.ShapeDtypeStruct((s * k1,), jnp.int32),
                  jax.ShapeDtypeStruct((s * k2,), jnp.int32),
                  jax.ShapeDtypeStruct((s * k3,), jnp.int32)),
        scratch_types=[pltpu.VMEM((n,), jnp.int32),
                       pltpu.VMEM((rpw * k1,), jnp.int32),
                       pltpu.VMEM((rpw * k2,), jnp.int32),
                       pltpu.VMEM((rpw * k3,), jnp.int32)],
    )
    def kern(pk_hbm, g1_hbm, g2_hbm, g3_hbm, rowbuf, ob1, ob2, ob3):
        wid = lax.axis_index("s") * _SC_NC + lax.axis_index("c")
        lanes = lax.iota(jnp.int32, 16)

        def row_body(r, _):
            row_g = wid * rpw + r
            pltpu.sync_copy(pk_hbm.at[row_g], rowbuf)

            def body(i, _c):
                v = rowbuf[pl.ds(i * 16, 16)]
                idxv = i * 16 + lanes
                cls = lax.shift_right_logical(v, 24)
                for b, (kk, ob) in enumerate(((k1, ob1), (k2, ob2),
                                              (k3, ob3))):
                    rk = lax.shift_right_logical(v, 8 * b) & 255
                    mw = (cls >= 3 - b) & (rk < kk)
                    plsc.store_scatter(ob, [r * kk + rk], idxv, mask=mw)
                return _c

            lax.fori_loop(0, nch, body, 0)
            return _

        lax.fori_loop(0, rpw, row_body, 0)
        base = wid * rpw
        pltpu.sync_copy(ob1, g1_hbm.at[pl.ds(base * k1, rpw * k1)])
        pltpu.sync_copy(ob2, g2_hbm.at[pl.ds(base * k2, rpw * k2)])
        pltpu.sync_copy(ob3, g3_hbm.at[pl.ds(base * k3, rpw * k3)])

    g1, g2, g3 = kern(packed)
    return (g1.reshape(s, k1), g2.reshape(s, k2), g3.reshape(s, k3))


def _sa_mlp_body(k, g_ref, *refs):
    (w1, b1, g1, be1, w2, b2, g2, be2, w3, b3, g3, be3, o_ref) = refs
    h = g_ref[...]
    y = lax.dot_general(h, w1[...], (((1,), (0,)), ((), ())),
                        preferred_element_type=F32)
    y = _bn_relu(y, b1[...], g1[...], be1[...])
    y = lax.dot_general(y.astype(BF16), w2[...], (((1,), (0,)), ((), ())),
                        preferred_element_type=F32)
    y = _bn_relu(y, b2[...], g2[...], be2[...])
    y = lax.dot_general(y.astype(BF16), w3[...], (((1,), (0,)), ((), ())),
                        preferred_element_type=F32)
    y = _bn_relu(y, b3[...], g3[...], be3[...])
    rows, c3 = y.shape
    o_ref[...] = jnp.max(y.reshape(rows // k, k, c3), axis=1)


def _sa_mlp(gp_bf, k, layers):
    rows, cin = gp_bf.shape
    s = rows // k
    c3 = layers[2]["w"].shape[0]
    rows_target = 1024 if c3 >= 512 else 4096
    blk_s = max(1, min(s, rows_target // k))
    args = [gp_bf]
    in_specs = [pl.BlockSpec((blk_s * k, cin), lambda i: (i, 0))]
    for lp in layers:
        co, ci = lp["w"].shape
        args += [lp["w"].T.astype(BF16), lp["b"].reshape(1, co),
                 lp["gamma"].reshape(1, co), lp["beta"].reshape(1, co)]
        in_specs += [pl.BlockSpec((ci, co), lambda i: (0, 0)),
                     pl.BlockSpec((1, co), lambda i: (0, 0)),
                     pl.BlockSpec((1, co), lambda i: (0, 0)),
                     pl.BlockSpec((1, co), lambda i: (0, 0))]
    return pl.pallas_call(
        functools.partial(_sa_mlp_body, k),
        grid=(s // blk_s,),
        in_specs=in_specs,
        out_specs=pl.BlockSpec((blk_s, c3), lambda i: (i, 0)),
        out_shape=jax.ShapeDtypeStruct((s, c3), F32),
    )(*args)


def _fp_body(x1_ref, x2t_ref, p1_ref, p2_ref, *refs):
    (w1a, w1b, b1, g1, be1, w2, b2, g2, be2, o_ref) = refs
    q = x1_ref[...]
    pt = x2t_ref[...]
    sq = (q[:, 0:1] * q[:, 0:1] + q[:, 1:2] * q[:, 1:2]
          + q[:, 2:3] * q[:, 2:3])
    sp = (pt[0:1] * pt[0:1] + pt[1:2] * pt[1:2] + pt[2:3] * pt[2:3])
    m = lax.dot_general(q.astype(BF16), pt.astype(BF16),
                        (((1,), (0,)), ((), ())), preferred_element_type=F32)
    d = (sq + sp) - 2.0 * m
    inf = F32(np.inf)
    t1 = jnp.min(d, axis=1, keepdims=True)
    t2 = jnp.min(jnp.where(d > t1, d, inf), axis=1, keepdims=True)
    t3 = jnp.min(jnp.where(d > t2, d, inf), axis=1, keepdims=True)
    recip = jnp.where(d <= t3, 1.0 / (d + F32(1e-8)), 0.0)
    w = recip / jnp.sum(recip, axis=1, keepdims=True)
    interp = lax.dot_general(w, p2_ref[...], (((1,), (0,)), ((), ())),
                             precision=lax.Precision.HIGHEST,
                             preferred_element_type=F32)
    y = (lax.dot_general(p1_ref[...], w1a[...], (((1,), (0,)), ((), ())),
                         preferred_element_type=F32)
         + lax.dot_general(interp.astype(BF16), w1b[...],
                           (((1,), (0,)), ((), ())),
                           preferred_element_type=F32))
    y = _bn_relu(y, b1[...], g1[...], be1[...])
    y = lax.dot_general(y.astype(BF16), w2[...], (((1,), (0,)), ((), ())),
                        preferred_element_type=F32)
    y = _bn_relu(y, b2[...], g2[...], be2[...])
    o_ref[...] = y


def _fp(xyz1, xyz2, points1, points2, layers):
    s1, s2 = xyz1.shape[0], xyz2.shape[0]
    c1, c2 = points1.shape[1], points2.shape[1]
    blk = min(s1, 512)
    l1, l2 = layers
    co1 = l1["w"].shape[0]
    co2 = l2["w"].shape[0]
    w1a = l1["w"][:, :c1].T.astype(BF16)
    w1b = l1["w"][:, c1:].T.astype(BF16)
    args = [xyz1, xyz2.T, points1.astype(BF16), points2,
            w1a, w1b, l1["b"].reshape(1, co1), l1["gamma"].reshape(1, co1),
            l1["beta"].reshape(1, co1),
            l2["w"].T.astype(BF16), l2["b"].reshape(1, co2),
            l2["gamma"].reshape(1, co2), l2["beta"].reshape(1, co2)]
    in_specs = [pl.BlockSpec((blk, 3), lambda i: (i, 0)),
                pl.BlockSpec((3, s2), lambda i: (0, 0)),
                pl.BlockSpec((blk, c1), lambda i: (i, 0)),
                pl.BlockSpec((s2, c2), lambda i: (0, 0)),
                pl.BlockSpec((c1, co1), lambda i: (0, 0)),
                pl.BlockSpec((c2, co1), lambda i: (0, 0)),
                pl.BlockSpec((1, co1), lambda i: (0, 0)),
                pl.BlockSpec((1, co1), lambda i: (0, 0)),
                pl.BlockSpec((1, co1), lambda i: (0, 0)),
                pl.BlockSpec((co1, co2), lambda i: (0, 0)),
                pl.BlockSpec((1, co2), lambda i: (0, 0)),
                pl.BlockSpec((1, co2), lambda i: (0, 0)),
                pl.BlockSpec((1, co2), lambda i: (0, 0))]
    return pl.pallas_call(
        _fp_body,
        grid=(s1 // blk,),
        in_specs=in_specs,
        out_specs=pl.BlockSpec((blk, co2), lambda i: (i, 0)),
        out_shape=jax.ShapeDtypeStruct((s1, co2), F32),
    )(*args)


def _head_body(x_ref, w1, b1, g1, be1, w2, b2, o_ref):
    y = lax.dot_general(x_ref[...], w1[...], (((1,), (0,)), ((), ())),
                        preferred_element_type=F32)
    y = _bn_relu(y, b1[...], g1[...], be1[...])
    z = lax.dot_general(y.astype(BF16), w2[...], (((1,), (0,)), ((), ())),
                        preferred_element_type=F32) + b2[...]
    o_ref[...] = 1.0 / (1.0 + jnp.exp(-z))


def _head(l0p, head):
    s = l0p.shape[0]
    blk = min(s, 1024)
    c1p = head["conv1"]
    c2p = head["conv2"]
    nc = c2p["w"].shape[0]
    args = [l0p.astype(BF16), c1p["w"].T.astype(BF16),
            c1p["b"].reshape(1, 128), c1p["gamma"].reshape(1, 128),
            c1p["beta"].reshape(1, 128),
            c2p["w"].T.astype(BF16), c2p["b"].reshape(1, nc)]
    in_specs = [pl.BlockSpec((blk, 128), lambda i: (i, 0)),
                pl.BlockSpec((128, 128), lambda i: (0, 0)),
                pl.BlockSpec((1, 128), lambda i: (0, 0)),
                pl.BlockSpec((1, 128), lambda i: (0, 0)),
                pl.BlockSpec((1, 128), lambda i: (0, 0)),
                pl.BlockSpec((128, nc), lambda i: (0, 0)),
                pl.BlockSpec((1, nc), lambda i: (0, 0))]
    return pl.pallas_call(
        _head_body,
        grid=(s // blk,),
        in_specs=in_specs,
        out_specs=pl.BlockSpec((blk, nc), lambda i: (i, 0)),
        out_shape=jax.ShapeDtypeStruct((s, nc), F32),
    )(*args)


def _sa(xyz, points, cfg, params):
    npoint, radii, nsamples, _, _ = cfg
    n = xyz.shape[0]
    _, new_xyz = _fps(xyz, npoint)
    packed, cnts = _distrank(new_xyz, xyz, radii)
    g1, g2, g3 = _sc_compact(packed, nsamples)
    feat = jnp.concatenate([points, xyz], axis=-1)
    cin = points.shape[-1]
    outs = []
    for bi, (gi_raw, k, layers) in enumerate(zip((g1, g2, g3), nsamples,
                                                 params)):
        cnt = cnts[:, bi:bi + 1]
        pad = jnp.where(cnt >= 1, gi_raw[:, :1], n - 1)
        slot = jnp.arange(k, dtype=jnp.int32)[None, :]
        gi = jnp.where(slot < cnt, gi_raw, pad)
        g = feat[gi]
        gp = jnp.concatenate(
            [g[..., :cin], g[..., cin:] - new_xyz[:, None, :]], axis=-1)
        gp_bf = gp.reshape(-1, cin + 3).astype(BF16)
        outs.append(_sa_mlp(gp_bf, k, layers))
    return new_xyz, jnp.concatenate(outs, axis=-1)


def kernel(xyz, params):
    x0 = xyz[0]
    l0_xyz, l0_points = x0[:, :3], x0[:, 3:]
    l1_xyz, l1_points = _sa(l0_xyz, l0_points, SA_CFGS[0], params["sa"][0])
    l2_xyz, l2_points = _sa(l1_xyz, l1_points, SA_CFGS[1], params["sa"][1])
    l3_xyz, l3_points = _sa(l2_xyz, l2_points, SA_CFGS[2], params["sa"][2])
    l4_xyz, l4_points = _sa(l3_xyz, l3_points, SA_CFGS[3], params["sa"][3])
    l3_points = _fp(l3_xyz, l4_xyz, l3_points, l4_points, params["fp"][0])
    l2_points = _fp(l2_xyz, l3_xyz, l2_points, l3_points, params["fp"][1])
    l1_points = _fp(l1_xyz, l2_xyz, l1_points, l2_points, params["fp"][2])
    l0p = _fp(l0_xyz, l1_xyz, l0_points, l1_points, params["fp"][3])
    y = _head(l0p, params["head"])
    return y[None], l0p[None]


# speedup vs baseline: 11.7943x; 1.5250x over previous
import functools

import jax
import jax.numpy as jnp
import numpy as np
from jax import lax
from jax.experimental import pallas as pl
from jax.experimental.pallas import tpu as pltpu
from jax.experimental.pallas import tpu_sc as plsc

B, N, NUM_CLASSES = 1, 4096, 13
BN_EPS = 1e-5
BN_DIV = np.float32(np.sqrt(1.0 + BN_EPS))
F32 = jnp.float32
BF16 = jnp.bfloat16

SA_CFGS = [
    (4096, [0.2, 0.4, 0.6], [32, 64, 64], 1, [[32, 32, 64], [64, 64, 128], [64, 96, 128]]),
    (1024, [0.4, 0.8, 1.6], [16, 32, 32], 320, [[64, 64, 128], [128, 128, 256], [128, 128, 256]]),
    (256, [0.8, 1.6, 3.2], [16, 32, 64], 640, [[128, 128, 256], [128, 128, 256], [128, 256, 256]]),
    (64, [1.6, 3.2, 6.4], [16, 32, 64], 768, [[256, 256, 512], [256, 256, 512], [256, 512, 1024]]),
]


def _bn_relu(y, b, gamma, beta):
    y = y + b
    y = y / BN_DIV * gamma + beta
    return jnp.maximum(y, 0.0)


def _fps_body(n, npoint, xr, yr, zr, idx_ref, nx_ref):
    n8 = n // 8
    X, Y, Z = xr[...], yr[...], zr[...]
    iota = (lax.broadcasted_iota(jnp.int32, (8, n8), 0) * n8
            + lax.broadcasted_iota(jnp.int32, (8, n8), 1))

    def body(t, carry):
        dist, far = carry
        oh = iota == far
        cx = jnp.sum(jnp.where(oh, X, 0.0))
        cy = jnp.sum(jnp.where(oh, Y, 0.0))
        cz = jnp.sum(jnp.where(oh, Z, 0.0))
        idx_ref[pl.ds(t, 1), :] = jnp.full((1, 1), far, jnp.int32)
        nx_ref[pl.ds(t, 1), :] = jnp.concatenate(
            [cx.reshape(1, 1), cy.reshape(1, 1), cz.reshape(1, 1)], axis=1)
        dx = X - cx
        dy = Y - cy
        dz = Z - cz
        d = dx * dx + dy * dy + dz * dz
        dist = jnp.minimum(dist, d)
        dmax = jnp.max(dist)
        far2 = jnp.min(jnp.where(dist == dmax, iota, n))
        return dist, far2

    lax.fori_loop(0, npoint, body,
                  (jnp.full((8, n8), 1e10, F32), jnp.int32(0)))


def _fps(xyz, npoint):
    n = xyz.shape[0]
    xr = xyz[:, 0].reshape(8, n // 8)
    yr = xyz[:, 1].reshape(8, n // 8)
    zr = xyz[:, 2].reshape(8, n // 8)
    idx, nx = pl.pallas_call(
        functools.partial(_fps_body, n, npoint),
        out_shape=(jax.ShapeDtypeStruct((npoint, 1), jnp.int32),
                   jax.ShapeDtypeStruct((npoint, 3), F32)),
    )(xr, yr, zr)
    return idx[:, 0], nx


def _dist_body(q_ref, pt_ref, d_ref):
    q = q_ref[...]
    pt = pt_ref[...]
    sq = (q[:, 0:1] * q[:, 0:1] + q[:, 1:2] * q[:, 1:2]
          + q[:, 2:3] * q[:, 2:3])
    sp = (pt[0:1] * pt[0:1] + pt[1:2] * pt[1:2] + pt[2:3] * pt[2:3])
    m = lax.dot_general(q.astype(BF16), pt.astype(BF16),
                        (((1,), (0,)), ((), ())),
                        preferred_element_type=F32)
    d_ref[...] = (sq + sp) - 2.0 * m


def _sqdist(a, b):
    s, n = a.shape[0], b.shape[0]
    blk = min(s, 512)
    return pl.pallas_call(
        _dist_body,
        grid=(s // blk,),
        in_specs=[pl.BlockSpec((blk, 3), lambda i: (i, 0)),
                  pl.BlockSpec((3, n), lambda i: (0, 0))],
        out_specs=pl.BlockSpec((blk, n), lambda i: (i, 0)),
        out_shape=jax.ShapeDtypeStruct((s, n), F32),
    )(a, b.T)


def _distrank_body(r2s, q_ref, pt_ref, p_ref, cnt_ref):
    q = q_ref[...]
    pt = pt_ref[...]
    sq = (q[:, 0:1] * q[:, 0:1] + q[:, 1:2] * q[:, 1:2]
          + q[:, 2:3] * q[:, 2:3])
    sp = (pt[0:1] * pt[0:1] + pt[1:2] * pt[1:2] + pt[2:3] * pt[2:3])
    m = lax.dot_general(q.astype(BF16), pt.astype(BF16),
                        (((1,), (0,)), ((), ())),
                        preferred_element_type=F32)
    d = (sq + sp) - 2.0 * m
    n = d.shape[1]
    ri = lax.broadcasted_iota(jnp.int32, (128, 128), 0)
    ci = lax.broadcasted_iota(jnp.int32, (128, 128), 1)
    u = (ri < ci).astype(BF16)
    blk = d.shape[0]
    bases = [jnp.zeros((blk, 1), F32) for _ in r2s]
    for c in range(n // 128):
        dch = d[:, c * 128:(c + 1) * 128]
        pch = jnp.zeros((blk, 128), jnp.int32)
        clsch = jnp.zeros((blk, 128), jnp.int32)
        for b, r2 in enumerate(r2s):
            mc = (dch <= r2).astype(F32)
            clsch = clsch + mc.astype(jnp.int32)
            excl = bases[b] + lax.dot_general(
                mc.astype(BF16), u, (((1,), (0,)), ((), ())),
                preferred_element_type=F32)
            bases[b] = bases[b] + jnp.sum(mc, axis=1, keepdims=True)
            pch = pch | (jnp.minimum(excl, 255.0).astype(jnp.int32) << (8 * b))
        p_ref[:, c * 128:(c + 1) * 128] = pch | (clsch << 24)
    cnt_ref[...] = jnp.concatenate(bases, axis=1).astype(jnp.int32)


def _distrank(a, b, radii):
    s, n = a.shape[0], b.shape[0]
    blk = min(s, 512)
    r2s = tuple(np.float32(r ** 2) for r in radii)
    return pl.pallas_call(
        functools.partial(_distrank_body, r2s),
        grid=(s // blk,),
        in_specs=[pl.BlockSpec((blk, 3), lambda i: (i, 0)),
                  pl.BlockSpec((3, n), lambda i: (0, 0))],
        out_specs=(pl.BlockSpec((blk, n), lambda i: (i, 0)),
                   pl.BlockSpec((blk, 3), lambda i: (i, 0))),
        out_shape=(jax.ShapeDtypeStruct((s, n), jnp.int32),
                   jax.ShapeDtypeStruct((s, 3), jnp.int32)),
    )(a, b.T)


_SC_NC = 2
_SC_NW = 32


def _sc_compact(packed, ks):
    s, n = packed.shape
    k1, k2, k3 = ks
    rpw = s // _SC_NW
    nch = n // 16
    mesh = plsc.VectorSubcoreMesh(core_axis_name="c", subcore_axis_name="s")

    @functools.partial(
        pl.kernel, mesh=mesh,
        compiler_params=pltpu.CompilerParams(needs_layout_passes=False),
        out_type=(jax.ShapeDtypeStruct((s * k1,), jnp.int32),
                  jax.ShapeDtypeStruct((s * k2,), jnp.int32),
                  jax.ShapeDtypeStruct((s * k3,), jnp.int32)),
        scratch_types=[pltpu.VMEM((n,), jnp.int32),
                       pltpu.VMEM((rpw * k1,), jnp.int32),
                       pltpu.VMEM((rpw * k2,), jnp.int32),
                       pltpu.VMEM((rpw * k3,), jnp.int32)],
    )
    def kern(pk_hbm, g1_hbm, g2_hbm, g3_hbm, rowbuf, ob1, ob2, ob3):
        wid = lax.axis_index("s") * _SC_NC + lax.axis_index("c")
        lanes = lax.iota(jnp.int32, 16)

        def row_body(r, _):
            row_g = wid * rpw + r
            pltpu.sync_copy(pk_hbm.at[row_g], rowbuf)

            def body(i, _c):
                v = rowbuf[pl.ds(i * 16, 16)]
                idxv = i * 16 + lanes
                cls = lax.shift_right_logical(v, 24)
                for b, (kk, ob) in enumerate(((k1, ob1), (k2, ob2),
                                              (k3, ob3))):
                    rk = lax.shift_right_logical(v, 8 * b) & 255
                    mw = (cls >= 3 - b) & (rk < kk)
                    plsc.store_scatter(ob, [r * kk + rk], idxv, mask=mw)
                return _c

            lax.fori_loop(0, nch, body, 0)
            return _

        lax.fori_loop(0, rpw, row_body, 0)
        base = wid * rpw
        pltpu.sync_copy(ob1, g1_hbm.at[pl.ds(base * k1, rpw * k1)])
        pltpu.sync_copy(ob2, g2_hbm.at[pl.ds(base * k2, rpw * k2)])
        pltpu.sync_copy(ob3, g3_hbm.at[pl.ds(base * k3, rpw * k3)])

    g1, g2, g3 = kern(packed)
    return (g1.reshape(s, k1), g2.reshape(s, k2), g3.reshape(s, k3))


def _sc_gather(tab, idx):
    n, dd = tab.shape
    bb = idx.shape[0]
    bpw = bb // _SC_NW
    ch = min(bpw, max(16, min(256, (400 * 1024 // (dd * 4)) // 16 * 16)))
    nch = bpw // ch
    mesh = plsc.VectorSubcoreMesh(core_axis_name="c", subcore_axis_name="s")

    @functools.partial(
        pl.kernel, mesh=mesh,
        compiler_params=pltpu.CompilerParams(use_tc_tiling_on_sc=False),
        out_type=jax.ShapeDtypeStruct((bb, dd), F32),
        scratch_types=[pltpu.VMEM((ch,), jnp.int32),
                       pltpu.VMEM((ch, dd), F32),
                       pltpu.SemaphoreType.DMA],
    )
    def kern(tab_hbm, idx_hbm, out_hbm, idxbuf, rowsbuf, sem):
        wid = lax.axis_index("s") * _SC_NC + lax.axis_index("c")
        base = wid * bpw

        def chunk_body(c, _):
            off = base + c * ch
            pltpu.sync_copy(idx_hbm.at[pl.ds(off, ch)], idxbuf)
            pltpu.async_copy(tab_hbm.at[idxbuf], rowsbuf, sem).wait()
            pltpu.sync_copy(rowsbuf, out_hbm.at[pl.ds(off, ch)])
            return _

        lax.fori_loop(0, nch, chunk_body, 0)

    return kern(tab, idx)


def _sa_mlp_body(k, g_ref, *refs):
    (w1, b1, g1, be1, w2, b2, g2, be2, w3, b3, g3, be3, o_ref) = refs
    h = g_ref[...]
    y = lax.dot_general(h, w1[...], (((1,), (0,)), ((), ())),
                        preferred_element_type=F32)
    y = _bn_relu(y, b1[...], g1[...], be1[...])
    y = lax.dot_general(y.astype(BF16), w2[...], (((1,), (0,)), ((), ())),
                        preferred_element_type=F32)
    y = _bn_relu(y, b2[...], g2[...], be2[...])
    y = lax.dot_general(y.astype(BF16), w3[...], (((1,), (0,)), ((), ())),
                        preferred_element_type=F32)
    y = _bn_relu(y, b3[...], g3[...], be3[...])
    rows, c3 = y.shape
    o_ref[...] = jnp.max(y.reshape(rows // k, k, c3), axis=1)


def _sa_mlp(gp_bf, k, layers):
    rows, cin = gp_bf.shape
    s = rows // k
    c3 = layers[2]["w"].shape[0]
    rows_target = 1024 if c3 >= 512 else 4096
    blk_s = max(1, min(s, rows_target // k))
    args = [gp_bf]
    in_specs = [pl.BlockSpec((blk_s * k, cin), lambda i: (i, 0))]
    for li, lp in enumerate(layers):
        co, ci = lp["w"].shape
        w = lp["w"]
        if li == 0 and ci < cin:
            w = jnp.concatenate([w, jnp.zeros((co, cin - ci), F32)], axis=1)
            ci = cin
        args += [w.T.astype(BF16), lp["b"].reshape(1, co),
                 lp["gamma"].reshape(1, co), lp["beta"].reshape(1, co)]
        in_specs += [pl.BlockSpec((ci, co), lambda i: (0, 0)),
                     pl.BlockSpec((1, co), lambda i: (0, 0)),
                     pl.BlockSpec((1, co), lambda i: (0, 0)),
                     pl.BlockSpec((1, co), lambda i: (0, 0))]
    return pl.pallas_call(
        functools.partial(_sa_mlp_body, k),
        grid=(s // blk_s,),
        in_specs=in_specs,
        out_specs=pl.BlockSpec((blk_s, c3), lambda i: (i, 0)),
        out_shape=jax.ShapeDtypeStruct((s, c3), F32),
    )(*args)


def _fp_body(x1_ref, x2t_ref, p1_ref, p2_ref, *refs):
    (w1a, w1b, b1, g1, be1, w2, b2, g2, be2, o_ref) = refs
    q = x1_ref[...]
    pt = x2t_ref[...]
    sq = (q[:, 0:1] * q[:, 0:1] + q[:, 1:2] * q[:, 1:2]
          + q[:, 2:3] * q[:, 2:3])
    sp = (pt[0:1] * pt[0:1] + pt[1:2] * pt[1:2] + pt[2:3] * pt[2:3])
    m = lax.dot_general(q.astype(BF16), pt.astype(BF16),
                        (((1,), (0,)), ((), ())), preferred_element_type=F32)
    d = (sq + sp) - 2.0 * m
    inf = F32(np.inf)
    t1 = jnp.min(d, axis=1, keepdims=True)
    t2 = jnp.min(jnp.where(d > t1, d, inf), axis=1, keepdims=True)
    t3 = jnp.min(jnp.where(d > t2, d, inf), axis=1, keepdims=True)
    recip = jnp.where(d <= t3, 1.0 / (d + F32(1e-8)), 0.0)
    w = recip / jnp.sum(recip, axis=1, keepdims=True)
    interp = lax.dot_general(w, p2_ref[...], (((1,), (0,)), ((), ())),
                             precision=lax.Precision.HIGHEST,
                             preferred_element_type=F32)
    y = (lax.dot_general(p1_ref[...], w1a[...], (((1,), (0,)), ((), ())),
                         preferred_element_type=F32)
         + lax.dot_general(interp.astype(BF16), w1b[...],
                           (((1,), (0,)), ((), ())),
                           preferred_element_type=F32))
    y = _bn_relu(y, b1[...], g1[...], be1[...])
    y = lax.dot_general(y.astype(BF16), w2[...], (((1,), (0,)), ((), ())),
                        preferred_element_type=F32)
    y = _bn_relu(y, b2[...], g2[...], be2[...])
    o_ref[...] = y


def _fp(xyz1, xyz2, points1, points2, layers):
    s1, s2 = xyz1.shape[0], xyz2.shape[0]
    c1, c2 = points1.shape[1], points2.shape[1]
    blk = min(s1, 512)
    l1, l2 = layers
    co1 = l1["w"].shape[0]
    co2 = l2["w"].shape[0]
    w1a = l1["w"][:, :c1].T.astype(BF16)
    w1b = l1["w"][:, c1:].T.astype(BF16)
    args = [xyz1, xyz2.T, points1.astype(BF16), points2,
            w1a, w1b, l1["b"].reshape(1, co1), l1["gamma"].reshape(1, co1),
            l1["beta"].reshape(1, co1),
            l2["w"].T.astype(BF16), l2["b"].reshape(1, co2),
            l2["gamma"].reshape(1, co2), l2["beta"].reshape(1, co2)]
    in_specs = [pl.BlockSpec((blk, 3), lambda i: (i, 0)),
                pl.BlockSpec((3, s2), lambda i: (0, 0)),
                pl.BlockSpec((blk, c1), lambda i: (i, 0)),
                pl.BlockSpec((s2, c2), lambda i: (0, 0)),
                pl.BlockSpec((c1, co1), lambda i: (0, 0)),
                pl.BlockSpec((c2, co1), lambda i: (0, 0)),
                pl.BlockSpec((1, co1), lambda i: (0, 0)),
                pl.BlockSpec((1, co1), lambda i: (0, 0)),
                pl.BlockSpec((1, co1), lambda i: (0, 0)),
                pl.BlockSpec((co1, co2), lambda i: (0, 0)),
                pl.BlockSpec((1, co2), lambda i: (0, 0)),
                pl.BlockSpec((1, co2), lambda i: (0, 0)),
                pl.BlockSpec((1, co2), lambda i: (0, 0))]
    return pl.pallas_call(
        _fp_body,
        grid=(s1 // blk,),
        in_specs=in_specs,
        out_specs=pl.BlockSpec((blk, co2), lambda i: (i, 0)),
        out_shape=jax.ShapeDtypeStruct((s1, co2), F32),
    )(*args)


def _head_body(x_ref, w1, b1, g1, be1, w2, b2, o_ref):
    y = lax.dot_general(x_ref[...], w1[...], (((1,), (0,)), ((), ())),
                        preferred_element_type=F32)
    y = _bn_relu(y, b1[...], g1[...], be1[...])
    z = lax.dot_general(y.astype(BF16), w2[...], (((1,), (0,)), ((), ())),
                        preferred_element_type=F32) + b2[...]
    o_ref[...] = 1.0 / (1.0 + jnp.exp(-z))


def _head(l0p, head):
    s = l0p.shape[0]
    blk = min(s, 1024)
    c1p = head["conv1"]
    c2p = head["conv2"]
    nc = c2p["w"].shape[0]
    args = [l0p.astype(BF16), c1p["w"].T.astype(BF16),
            c1p["b"].reshape(1, 128), c1p["gamma"].reshape(1, 128),
            c1p["beta"].reshape(1, 128),
            c2p["w"].T.astype(BF16), c2p["b"].reshape(1, nc)]
    in_specs = [pl.BlockSpec((blk, 128), lambda i: (i, 0)),
                pl.BlockSpec((128, 128), lambda i: (0, 0)),
                pl.BlockSpec((1, 128), lambda i: (0, 0)),
                pl.BlockSpec((1, 128), lambda i: (0, 0)),
                pl.BlockSpec((1, 128), lambda i: (0, 0)),
                pl.BlockSpec((128, nc), lambda i: (0, 0)),
                pl.BlockSpec((1, nc), lambda i: (0, 0))]
    return pl.pallas_call(
        _head_body,
        grid=(s // blk,),
        in_specs=in_specs,
        out_specs=pl.BlockSpec((blk, nc), lambda i: (i, 0)),
        out_shape=jax.ShapeDtypeStruct((s, nc), F32),
    )(*args)


def _sa(xyz, points, cfg, params):
    npoint, radii, nsamples, _, _ = cfg
    n = xyz.shape[0]
    _, new_xyz = _fps(xyz, npoint)
    packed, cnts = _distrank(new_xyz, xyz, radii)
    g1, g2, g3 = _sc_compact(packed, nsamples)
    cin = points.shape[-1]
    d16 = ((cin + 3 + 15) // 16) * 16
    feat = jnp.concatenate(
        [points, xyz, jnp.zeros((n, d16 - cin - 3), F32)], axis=-1)
    outs = []
    for bi, (gi_raw, k, layers) in enumerate(zip((g1, g2, g3), nsamples,
                                                 params)):
        cnt = cnts[:, bi:bi + 1]
        pad = jnp.where(cnt >= 1, gi_raw[:, :1], n - 1)
        slot = jnp.arange(k, dtype=jnp.int32)[None, :]
        gi = jnp.where(slot < cnt, gi_raw, pad)
        s = gi.shape[0]
        g = _sc_gather(feat, gi.reshape(-1)).reshape(s, k, d16)
        gp = jnp.concatenate(
            [g[..., :cin], g[..., cin:cin + 3] - new_xyz[:, None, :],
             g[..., cin + 3:]], axis=-1)
        gp_bf = gp.reshape(-1, d16).astype(BF16)
        outs.append(_sa_mlp(gp_bf, k, layers))
    return new_xyz, jnp.concatenate(outs, axis=-1)


def kernel(xyz, params):
    x0 = xyz[0]
    l0_xyz, l0_points = x0[:, :3], x0[:, 3:]
    l1_xyz, l1_points = _sa(l0_xyz, l0_points, SA_CFGS[0], params["sa"][0])
    l2_xyz, l2_points = _sa(l1_xyz, l1_points, SA_CFGS[1], params["sa"][1])
    l3_xyz, l3_points = _sa(l2_xyz, l2_points, SA_CFGS[2], params["sa"][2])
    l4_xyz, l4_points = _sa(l3_xyz, l3_points, SA_CFGS[3], params["sa"][3])
    l3_points = _fp(l3_xyz, l4_xyz, l3_points, l4_points, params["fp"][0])
    l2_points = _fp(l2_xyz, l3_xyz, l2_points, l3_points, params["fp"][1])
    l1_points = _fp(l1_xyz, l2_xyz, l1_points, l2_points, params["fp"][2])
    l0p = _fp(l0_xyz, l1_xyz, l0_points, l1_points, params["fp"][3])
    y = _head(l0p, params["head"])
    return y[None], l0p[None]


# speedup vs baseline: 11.7961x; 1.0002x over previous
import functools

import jax
import jax.numpy as jnp
import numpy as np
from jax import lax
from jax.experimental import pallas as pl
from jax.experimental.pallas import tpu as pltpu
from jax.experimental.pallas import tpu_sc as plsc

B, N, NUM_CLASSES = 1, 4096, 13
BN_EPS = 1e-5
BN_DIV = np.float32(np.sqrt(1.0 + BN_EPS))
F32 = jnp.float32
BF16 = jnp.bfloat16

SA_CFGS = [
    (4096, [0.2, 0.4, 0.6], [32, 64, 64], 1, [[32, 32, 64], [64, 64, 128], [64, 96, 128]]),
    (1024, [0.4, 0.8, 1.6], [16, 32, 32], 320, [[64, 64, 128], [128, 128, 256], [128, 128, 256]]),
    (256, [0.8, 1.6, 3.2], [16, 32, 64], 640, [[128, 128, 256], [128, 128, 256], [128, 256, 256]]),
    (64, [1.6, 3.2, 6.4], [16, 32, 64], 768, [[256, 256, 512], [256, 256, 512], [256, 512, 1024]]),
]


def _bn_relu(y, b, gamma, beta):
    y = y + b
    y = y / BN_DIV * gamma + beta
    return jnp.maximum(y, 0.0)


def _fps_body(n, npoint, xr, yr, zr, idx_ref, nx_ref):
    n8 = n // 8
    X, Y, Z = xr[...], yr[...], zr[...]
    iota = (lax.broadcasted_iota(jnp.int32, (8, n8), 0) * n8
            + lax.broadcasted_iota(jnp.int32, (8, n8), 1))

    def body(t, carry):
        dist, far = carry
        oh = iota == far
        cx = jnp.sum(jnp.where(oh, X, 0.0))
        cy = jnp.sum(jnp.where(oh, Y, 0.0))
        cz = jnp.sum(jnp.where(oh, Z, 0.0))
        idx_ref[pl.ds(t, 1), :] = jnp.full((1, 1), far, jnp.int32)
        nx_ref[pl.ds(t, 1), :] = jnp.concatenate(
            [cx.reshape(1, 1), cy.reshape(1, 1), cz.reshape(1, 1)], axis=1)
        dx = X - cx
        dy = Y - cy
        dz = Z - cz
        d = dx * dx + dy * dy + dz * dz
        dist = jnp.minimum(dist, d)
        dmax = jnp.max(dist)
        far2 = jnp.min(jnp.where(dist == dmax, iota, n))
        return dist, far2

    lax.fori_loop(0, npoint, body,
                  (jnp.full((8, n8), 1e10, F32), jnp.int32(0)))


def _fps(xyz, npoint):
    n = xyz.shape[0]
    xr = xyz[:, 0].reshape(8, n // 8)
    yr = xyz[:, 1].reshape(8, n // 8)
    zr = xyz[:, 2].reshape(8, n // 8)
    idx, nx = pl.pallas_call(
        functools.partial(_fps_body, n, npoint),
        out_shape=(jax.ShapeDtypeStruct((npoint, 1), jnp.int32),
                   jax.ShapeDtypeStruct((npoint, 3), F32)),
    )(xr, yr, zr)
    return idx[:, 0], nx


def _distrank_body(r2s, q_ref, pt_ref, p_ref, cnt_ref):
    q = q_ref[...]
    pt = pt_ref[...]
    sq = (q[:, 0:1] * q[:, 0:1] + q[:, 1:2] * q[:, 1:2]
          + q[:, 2:3] * q[:, 2:3])
    sp = (pt[0:1] * pt[0:1] + pt[1:2] * pt[1:2] + pt[2:3] * pt[2:3])
    m = lax.dot_general(q.astype(BF16), pt.astype(BF16),
                        (((1,), (0,)), ((), ())),
                        preferred_element_type=F32)
    d = (sq + sp) - 2.0 * m
    n = d.shape[1]
    ri = lax.broadcasted_iota(jnp.int32, (128, 128), 0)
    ci = lax.broadcasted_iota(jnp.int32, (128, 128), 1)
    u = (ri < ci).astype(BF16)
    blk = d.shape[0]
    bases = [jnp.zeros((blk, 1), F32) for _ in r2s]
    for c in range(n // 128):
        dch = d[:, c * 128:(c + 1) * 128]
        pch = jnp.zeros((blk, 128), jnp.int32)
        clsch = jnp.zeros((blk, 128), jnp.int32)
        for b, r2 in enumerate(r2s):
            mc = (dch <= r2).astype(F32)
            clsch = clsch + mc.astype(jnp.int32)
            excl = bases[b] + lax.dot_general(
                mc.astype(BF16), u, (((1,), (0,)), ((), ())),
                preferred_element_type=F32)
            bases[b] = bases[b] + jnp.sum(mc, axis=1, keepdims=True)
            pch = pch | (jnp.minimum(excl, 255.0).astype(jnp.int32) << (8 * b))
        p_ref[:, c * 128:(c + 1) * 128] = pch | (clsch << 24)
    cnt_ref[...] = jnp.concatenate(bases, axis=1).astype(jnp.int32)


def _distrank(a, b, radii):
    s, n = a.shape[0], b.shape[0]
    blk = min(s, 512)
    r2s = tuple(np.float32(r ** 2) for r in radii)
    return pl.pallas_call(
        functools.partial(_distrank_body, r2s),
        grid=(s // blk,),
        in_specs=[pl.BlockSpec((blk, 3), lambda i: (i, 0)),
                  pl.BlockSpec((3, n), lambda i: (0, 0))],
        out_specs=(pl.BlockSpec((blk, n), lambda i: (i, 0)),
                   pl.BlockSpec((blk, 3), lambda i: (i, 0))),
        out_shape=(jax.ShapeDtypeStruct((s, n), jnp.int32),
                   jax.ShapeDtypeStruct((s, 3), jnp.int32)),
    )(a, b.T)


_SC_NC = 2
_SC_NW = 32


def _sc_compact(packed, ks):
    s, n = packed.shape
    k1, k2, k3 = ks
    rpw = s // _SC_NW
    nch = n // 16
    mesh = plsc.VectorSubcoreMesh(core_axis_name="c", subcore_axis_name="s")

    @functools.partial(
        pl.kernel, mesh=mesh,
        compiler_params=pltpu.CompilerParams(needs_layout_passes=False),
        out_type=(jax.ShapeDtypeStruct((s * k1,), jnp.int32),
                  jax.ShapeDtypeStruct((s * k2,), jnp.int32),
                  jax.ShapeDtypeStruct((s * k3,), jnp.int32)),
        scratch_types=[pltpu.VMEM((n,), jnp.int32),
                       pltpu.VMEM((rpw * k1,), jnp.int32),
                       pltpu.VMEM((rpw * k2,), jnp.int32),
                       pltpu.VMEM((rpw * k3,), jnp.int32)],
    )
    def kern(pk_hbm, g1_hbm, g2_hbm, g3_hbm, rowbuf, ob1, ob2, ob3):
        wid = lax.axis_index("s") * _SC_NC + lax.axis_index("c")
        lanes = lax.iota(jnp.int32, 16)

        def row_body(r, _):
            row_g = wid * rpw + r
            pltpu.sync_copy(pk_hbm.at[row_g], rowbuf)

            def body(i, _c):
                v = rowbuf[pl.ds(i * 16, 16)]
                idxv = i * 16 + lanes
                cls = lax.shift_right_logical(v, 24)
                for b, (kk, ob) in enumerate(((k1, ob1), (k2, ob2),
                                              (k3, ob3))):
                    rk = lax.shift_right_logical(v, 8 * b) & 255
                    mw = (cls >= 3 - b) & (rk < kk)
                    plsc.store_scatter(ob, [r * kk + rk], idxv, mask=mw)
                return _c

            lax.fori_loop(0, nch, body, 0)
            return _

        lax.fori_loop(0, rpw, row_body, 0)
        base = wid * rpw
        pltpu.sync_copy(ob1, g1_hbm.at[pl.ds(base * k1, rpw * k1)])
        pltpu.sync_copy(ob2, g2_hbm.at[pl.ds(base * k2, rpw * k2)])
        pltpu.sync_copy(ob3, g3_hbm.at[pl.ds(base * k3, rpw * k3)])

    g1, g2, g3 = kern(packed)
    return (g1.reshape(s, k1), g2.reshape(s, k2), g3.reshape(s, k3))


def _sc_gather(tab, idx):
    n, dd = tab.shape
    bb = idx.shape[0]
    bpw = bb // _SC_NW
    ch = min(bpw, max(16, min(256, (400 * 1024 // (dd * 4)) // 16 * 16)))
    nch = bpw // ch
    mesh = plsc.VectorSubcoreMesh(core_axis_name="c", subcore_axis_name="s")

    @functools.partial(
        pl.kernel, mesh=mesh,
        compiler_params=pltpu.CompilerParams(use_tc_tiling_on_sc=False),
        out_type=jax.ShapeDtypeStruct((bb, dd), F32),
        scratch_types=[pltpu.VMEM((ch,), jnp.int32),
                       pltpu.VMEM((ch, dd), F32),
                       pltpu.SemaphoreType.DMA],
    )
    def kern(tab_hbm, idx_hbm, out_hbm, idxbuf, rowsbuf, sem):
        wid = lax.axis_index("s") * _SC_NC + lax.axis_index("c")
        base = wid * bpw

        def chunk_body(c, _):
            off = base + c * ch
            pltpu.sync_copy(idx_hbm.at[pl.ds(off, ch)], idxbuf)
            pltpu.async_copy(tab_hbm.at[idxbuf], rowsbuf, sem).wait()
            pltpu.sync_copy(rowsbuf, out_hbm.at[pl.ds(off, ch)])
            return _

        lax.fori_loop(0, nch, chunk_body, 0)

    return kern(tab, idx)


def _sa_mlp_body(k, g_ref, *refs):
    (w1, b1, g1, be1, w2, b2, g2, be2, w3, b3, g3, be3, o_ref) = refs
    h = g_ref[...]
    y = lax.dot_general(h, w1[...], (((1,), (0,)), ((), ())),
                        preferred_element_type=F32)
    y = _bn_relu(y, b1[...], g1[...], be1[...])
    y = lax.dot_general(y.astype(BF16), w2[...], (((1,), (0,)), ((), ())),
                        preferred_element_type=F32)
    y = _bn_relu(y, b2[...], g2[...], be2[...])
    y = lax.dot_general(y.astype(BF16), w3[...], (((1,), (0,)), ((), ())),
                        preferred_element_type=F32)
    y = _bn_relu(y, b3[...], g3[...], be3[...])
    rows, c3 = y.shape
    o_ref[...] = jnp.max(y.reshape(rows // k, k, c3), axis=1)


def _sa_mlp(gp_bf, k, layers):
    rows, cin = gp_bf.shape
    s = rows // k
    c3 = layers[2]["w"].shape[0]
    rows_target = 1024 if c3 >= 512 else 4096
    blk_s = max(1, min(s, rows_target // k))
    args = [gp_bf]
    in_specs = [pl.BlockSpec((blk_s * k, cin), lambda i: (i, 0))]
    for li, lp in enumerate(layers):
        co, ci = lp["w"].shape
        w = lp["w"]
        if li == 0 and ci < cin:
            w = jnp.concatenate([w, jnp.zeros((co, cin - ci), F32)], axis=1)
            ci = cin
        args += [w.T.astype(BF16), lp["b"].reshape(1, co),
                 lp["gamma"].reshape(1, co), lp["beta"].reshape(1, co)]
        in_specs += [pl.BlockSpec((ci, co), lambda i: (0, 0)),
                     pl.BlockSpec((1, co), lambda i: (0, 0)),
                     pl.BlockSpec((1, co), lambda i: (0, 0)),
                     pl.BlockSpec((1, co), lambda i: (0, 0))]
    return pl.pallas_call(
        functools.partial(_sa_mlp_body, k),
        grid=(s // blk_s,),
        in_specs=in_specs,
        out_specs=pl.BlockSpec((blk_s, c3), lambda i: (i, 0)),
        out_shape=jax.ShapeDtypeStruct((s, c3), F32),
    )(*args)


def _fp_body(x1_ref, x2t_ref, p1_ref, p2_ref, *refs):
    (w1a, w1b, b1, g1, be1, w2, b2, g2, be2, o_ref) = refs
    q = x1_ref[...]
    pt = x2t_ref[...]
    sq = (q[:, 0:1] * q[:, 0:1] + q[:, 1:2] * q[:, 1:2]
          + q[:, 2:3] * q[:, 2:3])
    sp = (pt[0:1] * pt[0:1] + pt[1:2] * pt[1:2] + pt[2:3] * pt[2:3])
    m = lax.dot_general(q.astype(BF16), pt.astype(BF16),
                        (((1,), (0,)), ((), ())), preferred_element_type=F32)
    d = (sq + sp) - 2.0 * m
    inf = F32(np.inf)
    t1 = jnp.min(d, axis=1, keepdims=True)
    t2 = jnp.min(jnp.where(d > t1, d, inf), axis=1, keepdims=True)
    t3 = jnp.min(jnp.where(d > t2, d, inf), axis=1, keepdims=True)
    recip = jnp.where(d <= t3, 1.0 / (d + F32(1e-8)), 0.0)
    w = recip / jnp.sum(recip, axis=1, keepdims=True)
    interp = lax.dot_general(w, p2_ref[...], (((1,), (0,)), ((), ())),
                             precision=lax.Precision.HIGHEST,
                             preferred_element_type=F32)
    y = (lax.dot_general(p1_ref[...], w1a[...], (((1,), (0,)), ((), ())),
                         preferred_element_type=F32)
         + lax.dot_general(interp.astype(BF16), w1b[...],
                           (((1,), (0,)), ((), ())),
                           preferred_element_type=F32))
    y = _bn_relu(y, b1[...], g1[...], be1[...])
    y = lax.dot_general(y.astype(BF16), w2[...], (((1,), (0,)), ((), ())),
                        preferred_element_type=F32)
    y = _bn_relu(y, b2[...], g2[...], be2[...])
    o_ref[...] = y


def _fp(xyz1, xyz2, points1, points2, layers):
    s1, s2 = xyz1.shape[0], xyz2.shape[0]
    c1, c2 = points1.shape[1], points2.shape[1]
    blk = min(s1, 512)
    l1, l2 = layers
    co1 = l1["w"].shape[0]
    co2 = l2["w"].shape[0]
    w1a = l1["w"][:, :c1].T.astype(BF16)
    w1b = l1["w"][:, c1:].T.astype(BF16)
    args = [xyz1, xyz2.T, points1.astype(BF16), points2,
            w1a, w1b, l1["b"].reshape(1, co1), l1["gamma"].reshape(1, co1),
            l1["beta"].reshape(1, co1),
            l2["w"].T.astype(BF16), l2["b"].reshape(1, co2),
            l2["gamma"].reshape(1, co2), l2["beta"].reshape(1, co2)]
    in_specs = [pl.BlockSpec((blk, 3), lambda i: (i, 0)),
                pl.BlockSpec((3, s2), lambda i: (0, 0)),
                pl.BlockSpec((blk, c1), lambda i: (i, 0)),
                pl.BlockSpec((s2, c2), lambda i: (0, 0)),
                pl.BlockSpec((c1, co1), lambda i: (0, 0)),
                pl.BlockSpec((c2, co1), lambda i: (0, 0)),
                pl.BlockSpec((1, co1), lambda i: (0, 0)),
                pl.BlockSpec((1, co1), lambda i: (0, 0)),
                pl.BlockSpec((1, co1), lambda i: (0, 0)),
                pl.BlockSpec((co1, co2), lambda i: (0, 0)),
                pl.BlockSpec((1, co2), lambda i: (0, 0)),
                pl.BlockSpec((1, co2), lambda i: (0, 0)),
                pl.BlockSpec((1, co2), lambda i: (0, 0))]
    return pl.pallas_call(
        _fp_body,
        grid=(s1 // blk,),
        in_specs=in_specs,
        out_specs=pl.BlockSpec((blk, co2), lambda i: (i, 0)),
        out_shape=jax.ShapeDtypeStruct((s1, co2), F32),
    )(*args)


def _head_body(x_ref, w1, b1, g1, be1, w2, b2, o_ref):
    y = lax.dot_general(x_ref[...], w1[...], (((1,), (0,)), ((), ())),
                        preferred_element_type=F32)
    y = _bn_relu(y, b1[...], g1[...], be1[...])
    z = lax.dot_general(y.astype(BF16), w2[...], (((1,), (0,)), ((), ())),
                        preferred_element_type=F32) + b2[...]
    o_ref[...] = 1.0 / (1.0 + jnp.exp(-z))


def _head(l0p, head):
    s = l0p.shape[0]
    blk = min(s, 1024)
    c1p = head["conv1"]
    c2p = head["conv2"]
    nc = c2p["w"].shape[0]
    args = [l0p.astype(BF16), c1p["w"].T.astype(BF16),
            c1p["b"].reshape(1, 128), c1p["gamma"].reshape(1, 128),
            c1p["beta"].reshape(1, 128),
            c2p["w"].T.astype(BF16), c2p["b"].reshape(1, nc)]
    in_specs = [pl.BlockSpec((blk, 128), lambda i: (i, 0)),
                pl.BlockSpec((128, 128), lambda i: (0, 0)),
                pl.BlockSpec((1, 128), lambda i: (0, 0)),
                pl.BlockSpec((1, 128), lambda i: (0, 0)),
                pl.BlockSpec((1, 128), lambda i: (0, 0)),
                pl.BlockSpec((128, nc), lambda i: (0, 0)),
                pl.BlockSpec((1, nc), lambda i: (0, 0))]
    return pl.pallas_call(
        _head_body,
        grid=(s // blk,),
        in_specs=in_specs,
        out_specs=pl.BlockSpec((blk, nc), lambda i: (i, 0)),
        out_shape=jax.ShapeDtypeStruct((s, nc), F32),
    )(*args)


def _sa(xyz, points, cfg, params):
    npoint, radii, nsamples, _, _ = cfg
    n = xyz.shape[0]
    _, new_xyz = _fps(xyz, npoint)
    packed, cnts = _distrank(new_xyz, xyz, radii)
    g1, g2, g3 = _sc_compact(packed, nsamples)
    cin = points.shape[-1]
    d16 = ((cin + 3 + 15) // 16) * 16
    feat = jnp.concatenate(
        [points, xyz, jnp.zeros((n, d16 - cin - 3), F32)], axis=-1)
    outs = []
    for bi, (gi_raw, k, layers) in enumerate(zip((g1, g2, g3), nsamples,
                                                 params)):
        cnt = cnts[:, bi:bi + 1]
        pad = jnp.where(cnt >= 1, gi_raw[:, :1], n - 1)
        slot = jnp.arange(k, dtype=jnp.int32)[None, :]
        gi = jnp.where(slot < cnt, gi_raw, pad)
        s = gi.shape[0]
        g = _sc_gather(feat, gi.reshape(-1)).reshape(s, k, d16)
        gp = jnp.concatenate(
            [g[..., :cin], g[..., cin:cin + 3] - new_xyz[:, None, :],
             g[..., cin + 3:]], axis=-1)
        gp_bf = gp.reshape(-1, d16).astype(BF16)
        outs.append(_sa_mlp(gp_bf, k, layers))
    return new_xyz, jnp.concatenate(outs, axis=-1)


def kernel(xyz, params):
    x0 = xyz[0]
    l0_xyz, l0_points = x0[:, :3], x0[:, 3:]
    l1_xyz, l1_points = _sa(l0_xyz, l0_points, SA_CFGS[0], params["sa"][0])
    l2_xyz, l2_points = _sa(l1_xyz, l1_points, SA_CFGS[1], params["sa"][1])
    l3_xyz, l3_points = _sa(l2_xyz, l2_points, SA_CFGS[2], params["sa"][2])
    l4_xyz, l4_points = _sa(l3_xyz, l3_points, SA_CFGS[3], params["sa"][3])
    l3_points = _fp(l3_xyz, l4_xyz, l3_points, l4_points, params["fp"][0])
    l2_points = _fp(l2_xyz, l3_xyz, l2_points, l3_points, params["fp"][1])
    l1_points = _fp(l1_xyz, l2_xyz, l1_points, l2_points, params["fp"][2])
    l0p = _fp(l0_xyz, l1_xyz, l0_points, l1_points, params["fp"][3])
    y = _head(l0p, params["head"])
    return y[None], l0p[None]


# speedup vs baseline: 12.0364x; 1.0204x over previous
import functools

import jax
import jax.numpy as jnp
import numpy as np
from jax import lax
from jax.experimental import pallas as pl
from jax.experimental.pallas import tpu as pltpu
from jax.experimental.pallas import tpu_sc as plsc

B, N, NUM_CLASSES = 1, 4096, 13
BN_EPS = 1e-5
BN_DIV = np.float32(np.sqrt(1.0 + BN_EPS))
F32 = jnp.float32
BF16 = jnp.bfloat16

SA_CFGS = [
    (4096, [0.2, 0.4, 0.6], [32, 64, 64], 1, [[32, 32, 64], [64, 64, 128], [64, 96, 128]]),
    (1024, [0.4, 0.8, 1.6], [16, 32, 32], 320, [[64, 64, 128], [128, 128, 256], [128, 128, 256]]),
    (256, [0.8, 1.6, 3.2], [16, 32, 64], 640, [[128, 128, 256], [128, 128, 256], [128, 256, 256]]),
    (64, [1.6, 3.2, 6.4], [16, 32, 64], 768, [[256, 256, 512], [256, 256, 512], [256, 512, 1024]]),
]


def _bn_relu(y, b, gamma, beta):
    y = y + b
    y = y / BN_DIV * gamma + beta
    return jnp.maximum(y, 0.0)


def _fps_body(n, npoint, xr, yr, zr, x3_ref, idx_ref, nx_ref):
    n8 = n // 8
    X, Y, Z = xr[...], yr[...], zr[...]
    iota = (lax.broadcasted_iota(jnp.int32, (8, n8), 0) * n8
            + lax.broadcasted_iota(jnp.int32, (8, n8), 1))

    def body(t, carry):
        dist, far = carry
        crow = x3_ref[pl.ds(far, 1), :]
        cx = crow[0, 0]
        cy = crow[0, 1]
        cz = crow[0, 2]
        idx_ref[pl.ds(t, 1), :] = jnp.full((1, 1), far, jnp.int32)
        nx_ref[pl.ds(t, 1), :] = crow
        dx = X - cx
        dy = Y - cy
        dz = Z - cz
        d = dx * dx + dy * dy + dz * dz
        dist = jnp.minimum(dist, d)
        dmax = jnp.max(dist)
        far2 = jnp.min(jnp.where(dist == dmax, iota, n))
        return dist, far2

    lax.fori_loop(0, npoint, body,
                  (jnp.full((8, n8), 1e10, F32), jnp.int32(0)))


def _fps(xyz, npoint):
    n = xyz.shape[0]
    xr = xyz[:, 0].reshape(8, n // 8)
    yr = xyz[:, 1].reshape(8, n // 8)
    zr = xyz[:, 2].reshape(8, n // 8)
    idx, nx = pl.pallas_call(
        functools.partial(_fps_body, n, npoint),
        out_shape=(jax.ShapeDtypeStruct((npoint, 1), jnp.int32),
                   jax.ShapeDtypeStruct((npoint, 3), F32)),
    )(xr, yr, zr, xyz)
    return idx[:, 0], nx


def _distrank_body(r2s, q_ref, pt_ref, p_ref, cnt_ref):
    q = q_ref[...]
    pt = pt_ref[...]
    sq = (q[:, 0:1] * q[:, 0:1] + q[:, 1:2] * q[:, 1:2]
          + q[:, 2:3] * q[:, 2:3])
    sp = (pt[0:1] * pt[0:1] + pt[1:2] * pt[1:2] + pt[2:3] * pt[2:3])
    m = lax.dot_general(q.astype(BF16), pt.astype(BF16),
                        (((1,), (0,)), ((), ())),
                        preferred_element_type=F32)
    d = (sq + sp) - 2.0 * m
    n = d.shape[1]
    ri = lax.broadcasted_iota(jnp.int32, (128, 128), 0)
    ci = lax.broadcasted_iota(jnp.int32, (128, 128), 1)
    u = (ri < ci).astype(BF16)
    blk = d.shape[0]
    bases = [jnp.zeros((blk, 1), F32) for _ in r2s]
    for c in range(n // 128):
        dch = d[:, c * 128:(c + 1) * 128]
        pch = jnp.zeros((blk, 128), jnp.int32)
        clsch = jnp.zeros((blk, 128), jnp.int32)
        for b, r2 in enumerate(r2s):
            mc = (dch <= r2).astype(F32)
            clsch = clsch + mc.astype(jnp.int32)
            excl = bases[b] + lax.dot_general(
                mc.astype(BF16), u, (((1,), (0,)), ((), ())),
                preferred_element_type=F32)
            bases[b] = bases[b] + jnp.sum(mc, axis=1, keepdims=True)
            pch = pch | (jnp.minimum(excl, 255.0).astype(jnp.int32) << (8 * b))
        p_ref[:, c * 128:(c + 1) * 128] = pch | (clsch << 24)
    cnt_ref[...] = jnp.concatenate(bases, axis=1).astype(jnp.int32)


def _distrank(a, b, radii):
    s, n = a.shape[0], b.shape[0]
    blk = min(s, 512)
    r2s = tuple(np.float32(r ** 2) for r in radii)
    return pl.pallas_call(
        functools.partial(_distrank_body, r2s),
        grid=(s // blk,),
        in_specs=[pl.BlockSpec((blk, 3), lambda i: (i, 0)),
                  pl.BlockSpec((3, n), lambda i: (0, 0))],
        out_specs=(pl.BlockSpec((blk, n), lambda i: (i, 0)),
                   pl.BlockSpec((blk, 3), lambda i: (i, 0))),
        out_shape=(jax.ShapeDtypeStruct((s, n), jnp.int32),
                   jax.ShapeDtypeStruct((s, 3), jnp.int32)),
    )(a, b.T)


_SC_NC = 2
_SC_NW = 32


def _sc_compact(packed, ks):
    s, n = packed.shape
    k1, k2, k3 = ks
    rpw = s // _SC_NW
    nch = n // 16
    mesh = plsc.VectorSubcoreMesh(core_axis_name="c", subcore_axis_name="s")

    @functools.partial(
        pl.kernel, mesh=mesh,
        compiler_params=pltpu.CompilerParams(needs_layout_passes=False),
        out_type=(jax.ShapeDtypeStruct((s * k1,), jnp.int32),
                  jax.ShapeDtypeStruct((s * k2,), jnp.int32),
                  jax.ShapeDtypeStruct((s * k3,), jnp.int32)),
        scratch_types=[pltpu.VMEM((n,), jnp.int32),
                       pltpu.VMEM((rpw * k1,), jnp.int32),
                       pltpu.VMEM((rpw * k2,), jnp.int32),
                       pltpu.VMEM((rpw * k3,), jnp.int32)],
    )
    def kern(pk_hbm, g1_hbm, g2_hbm, g3_hbm, rowbuf, ob1, ob2, ob3):
        wid = lax.axis_index("s") * _SC_NC + lax.axis_index("c")
        lanes = lax.iota(jnp.int32, 16)

        def row_body(r, _):
            row_g = wid * rpw + r
            pltpu.sync_copy(pk_hbm.at[row_g], rowbuf)

            def body(i, _c):
                v = rowbuf[pl.ds(i * 16, 16)]
                idxv = i * 16 + lanes
                cls = lax.shift_right_logical(v, 24)
                for b, (kk, ob) in enumerate(((k1, ob1), (k2, ob2),
                                              (k3, ob3))):
                    rk = lax.shift_right_logical(v, 8 * b) & 255
                    mw = (cls >= 3 - b) & (rk < kk)
                    plsc.store_scatter(ob, [r * kk + rk], idxv, mask=mw)
                return _c

            lax.fori_loop(0, nch, body, 0)
            return _

        lax.fori_loop(0, rpw, row_body, 0)
        base = wid * rpw
        pltpu.sync_copy(ob1, g1_hbm.at[pl.ds(base * k1, rpw * k1)])
        pltpu.sync_copy(ob2, g2_hbm.at[pl.ds(base * k2, rpw * k2)])
        pltpu.sync_copy(ob3, g3_hbm.at[pl.ds(base * k3, rpw * k3)])

    g1, g2, g3 = kern(packed)
    return (g1.reshape(s, k1), g2.reshape(s, k2), g3.reshape(s, k3))


def _sc_gather(tab, idx):
    n, dd = tab.shape
    bb = idx.shape[0]
    bpw = bb // _SC_NW
    ch = min(bpw, max(16, min(256, (400 * 1024 // (dd * 4)) // 16 * 16)))
    nch = bpw // ch
    mesh = plsc.VectorSubcoreMesh(core_axis_name="c", subcore_axis_name="s")

    @functools.partial(
        pl.kernel, mesh=mesh,
        compiler_params=pltpu.CompilerParams(use_tc_tiling_on_sc=False),
        out_type=jax.ShapeDtypeStruct((bb, dd), F32),
        scratch_types=[pltpu.VMEM((ch,), jnp.int32),
                       pltpu.VMEM((ch, dd), F32),
                       pltpu.SemaphoreType.DMA],
    )
    def kern(tab_hbm, idx_hbm, out_hbm, idxbuf, rowsbuf, sem):
        wid = lax.axis_index("s") * _SC_NC + lax.axis_index("c")
        base = wid * bpw

        def chunk_body(c, _):
            off = base + c * ch
            pltpu.sync_copy(idx_hbm.at[pl.ds(off, ch)], idxbuf)
            pltpu.async_copy(tab_hbm.at[idxbuf], rowsbuf, sem).wait()
            pltpu.sync_copy(rowsbuf, out_hbm.at[pl.ds(off, ch)])
            return _

        lax.fori_loop(0, nch, chunk_body, 0)

    return kern(tab, idx)


def _sa_mlp_body(k, g_ref, *refs):
    (w1, b1, g1, be1, w2, b2, g2, be2, w3, b3, g3, be3, o_ref) = refs
    h = g_ref[...]
    y = lax.dot_general(h, w1[...], (((1,), (0,)), ((), ())),
                        preferred_element_type=F32)
    y = _bn_relu(y, b1[...], g1[...], be1[...])
    y = lax.dot_general(y.astype(BF16), w2[...], (((1,), (0,)), ((), ())),
                        preferred_element_type=F32)
    y = _bn_relu(y, b2[...], g2[...], be2[...])
    y = lax.dot_general(y.astype(BF16), w3[...], (((1,), (0,)), ((), ())),
                        preferred_element_type=F32)
    y = _bn_relu(y, b3[...], g3[...], be3[...])
    rows, c3 = y.shape
    o_ref[...] = jnp.max(y.reshape(rows // k, k, c3), axis=1)


def _sa_mlp(gp_bf, k, layers):
    rows, cin = gp_bf.shape
    s = rows // k
    c3 = layers[2]["w"].shape[0]
    rows_target = 1024 if c3 >= 512 else 4096
    blk_s = max(1, min(s, rows_target // k))
    args = [gp_bf]
    in_specs = [pl.BlockSpec((blk_s * k, cin), lambda i: (i, 0))]
    for li, lp in enumerate(layers):
        co, ci = lp["w"].shape
        w = lp["w"]
        if li == 0 and ci < cin:
            w = jnp.concatenate([w, jnp.zeros((co, cin - ci), F32)], axis=1)
            ci = cin
        args += [w.T.astype(BF16), lp["b"].reshape(1, co),
                 lp["gamma"].reshape(1, co), lp["beta"].reshape(1, co)]
        in_specs += [pl.BlockSpec((ci, co), lambda i: (0, 0)),
                     pl.BlockSpec((1, co), lambda i: (0, 0)),
                     pl.BlockSpec((1, co), lambda i: (0, 0)),
                     pl.BlockSpec((1, co), lambda i: (0, 0))]
    return pl.pallas_call(
        functools.partial(_sa_mlp_body, k),
        grid=(s // blk_s,),
        in_specs=in_specs,
        out_specs=pl.BlockSpec((blk_s, c3), lambda i: (i, 0)),
        out_shape=jax.ShapeDtypeStruct((s, c3), F32),
    )(*args)


def _fp_body(x1_ref, x2t_ref, p1_ref, p2_ref, *refs):
    (w1a, w1b, b1, g1, be1, w2, b2, g2, be2, o_ref) = refs
    q = x1_ref[...]
    pt = x2t_ref[...]
    sq = (q[:, 0:1] * q[:, 0:1] + q[:, 1:2] * q[:, 1:2]
          + q[:, 2:3] * q[:, 2:3])
    sp = (pt[0:1] * pt[0:1] + pt[1:2] * pt[1:2] + pt[2:3] * pt[2:3])
    m = lax.dot_general(q.astype(BF16), pt.astype(BF16),
                        (((1,), (0,)), ((), ())), preferred_element_type=F32)
    d = (sq + sp) - 2.0 * m
    inf = F32(np.inf)
    t1 = jnp.min(d, axis=1, keepdims=True)
    t2 = jnp.min(jnp.where(d > t1, d, inf), axis=1, keepdims=True)
    t3 = jnp.min(jnp.where(d > t2, d, inf), axis=1, keepdims=True)
    recip = jnp.where(d <= t3, 1.0 / (d + F32(1e-8)), 0.0)
    w = recip / jnp.sum(recip, axis=1, keepdims=True)
    interp = lax.dot_general(w, p2_ref[...], (((1,), (0,)), ((), ())),
                             precision=lax.Precision.HIGHEST,
                             preferred_element_type=F32)
    y = (lax.dot_general(p1_ref[...], w1a[...], (((1,), (0,)), ((), ())),
                         preferred_element_type=F32)
         + lax.dot_general(interp.astype(BF16), w1b[...],
                           (((1,), (0,)), ((), ())),
                           preferred_element_type=F32))
    y = _bn_relu(y, b1[...], g1[...], be1[...])
    y = lax.dot_general(y.astype(BF16), w2[...], (((1,), (0,)), ((), ())),
                        preferred_element_type=F32)
    y = _bn_relu(y, b2[...], g2[...], be2[...])
    o_ref[...] = y


def _fp(xyz1, xyz2, points1, points2, layers):
    s1, s2 = xyz1.shape[0], xyz2.shape[0]
    c1, c2 = points1.shape[1], points2.shape[1]
    blk = min(s1, 512)
    l1, l2 = layers
    co1 = l1["w"].shape[0]
    co2 = l2["w"].shape[0]
    w1a = l1["w"][:, :c1].T.astype(BF16)
    w1b = l1["w"][:, c1:].T.astype(BF16)
    args = [xyz1, xyz2.T, points1.astype(BF16), points2,
            w1a, w1b, l1["b"].reshape(1, co1), l1["gamma"].reshape(1, co1),
            l1["beta"].reshape(1, co1),
            l2["w"].T.astype(BF16), l2["b"].reshape(1, co2),
            l2["gamma"].reshape(1, co2), l2["beta"].reshape(1, co2)]
    in_specs = [pl.BlockSpec((blk, 3), lambda i: (i, 0)),
                pl.BlockSpec((3, s2), lambda i: (0, 0)),
                pl.BlockSpec((blk, c1), lambda i: (i, 0)),
                pl.BlockSpec((s2, c2), lambda i: (0, 0)),
                pl.BlockSpec((c1, co1), lambda i: (0, 0)),
                pl.BlockSpec((c2, co1), lambda i: (0, 0)),
                pl.BlockSpec((1, co1), lambda i: (0, 0)),
                pl.BlockSpec((1, co1), lambda i: (0, 0)),
                pl.BlockSpec((1, co1), lambda i: (0, 0)),
                pl.BlockSpec((co1, co2), lambda i: (0, 0)),
                pl.BlockSpec((1, co2), lambda i: (0, 0)),
                pl.BlockSpec((1, co2), lambda i: (0, 0)),
                pl.BlockSpec((1, co2), lambda i: (0, 0))]
    return pl.pallas_call(
        _fp_body,
        grid=(s1 // blk,),
        in_specs=in_specs,
        out_specs=pl.BlockSpec((blk, co2), lambda i: (i, 0)),
        out_shape=jax.ShapeDtypeStruct((s1, co2), F32),
    )(*args)


def _head_body(x_ref, w1, b1, g1, be1, w2, b2, o_ref):
    y = lax.dot_general(x_ref[...], w1[...], (((1,), (0,)), ((), ())),
                        preferred_element_type=F32)
    y = _bn_relu(y, b1[...], g1[...], be1[...])
    z = lax.dot_general(y.astype(BF16), w2[...], (((1,), (0,)), ((), ())),
                        preferred_element_type=F32) + b2[...]
    o_ref[...] = 1.0 / (1.0 + jnp.exp(-z))


def _head(l0p, head):
    s = l0p.shape[0]
    blk = min(s, 1024)
    c1p = head["conv1"]
    c2p = head["conv2"]
    nc = c2p["w"].shape[0]
    args = [l0p.astype(BF16), c1p["w"].T.astype(BF16),
            c1p["b"].reshape(1, 128), c1p["gamma"].reshape(1, 128),
            c1p["beta"].reshape(1, 128),
            c2p["w"].T.astype(BF16), c2p["b"].reshape(1, nc)]
    in_specs = [pl.BlockSpec((blk, 128), lambda i: (i, 0)),
                pl.BlockSpec((128, 128), lambda i: (0, 0)),
                pl.BlockSpec((1, 128), lambda i: (0, 0)),
                pl.BlockSpec((1, 128), lambda i: (0, 0)),
                pl.BlockSpec((1, 128), lambda i: (0, 0)),
                pl.BlockSpec((128, nc), lambda i: (0, 0)),
                pl.BlockSpec((1, nc), lambda i: (0, 0))]
    return pl.pallas_call(
        _head_body,
        grid=(s // blk,),
        in_specs=in_specs,
        out_specs=pl.BlockSpec((blk, nc), lambda i: (i, 0)),
        out_shape=jax.ShapeDtypeStruct((s, nc), F32),
    )(*args)


def _sa(xyz, points, cfg, params):
    npoint, radii, nsamples, _, _ = cfg
    n = xyz.shape[0]
    _, new_xyz = _fps(xyz, npoint)
    packed, cnts = _distrank(new_xyz, xyz, radii)
    g1, g2, g3 = _sc_compact(packed, nsamples)
    cin = points.shape[-1]
    d16 = ((cin + 3 + 15) // 16) * 16
    feat = jnp.concatenate(
        [points, xyz, jnp.zeros((n, d16 - cin - 3), F32)], axis=-1)
    outs = []
    for bi, (gi_raw, k, layers) in enumerate(zip((g1, g2, g3), nsamples,
                                                 params)):
        cnt = cnts[:, bi:bi + 1]
        pad = jnp.where(cnt >= 1, gi_raw[:, :1], n - 1)
        slot = jnp.arange(k, dtype=jnp.int32)[None, :]
        gi = jnp.where(slot < cnt, gi_raw, pad)
        s = gi.shape[0]
        g = _sc_gather(feat, gi.reshape(-1)).reshape(s, k, d16)
        gp = jnp.concatenate(
            [g[..., :cin], g[..., cin:cin + 3] - new_xyz[:, None, :],
             g[..., cin + 3:]], axis=-1)
        gp_bf = gp.reshape(-1, d16).astype(BF16)
        outs.append(_sa_mlp(gp_bf, k, layers))
    return new_xyz, jnp.concatenate(outs, axis=-1)


def kernel(xyz, params):
    x0 = xyz[0]
    l0_xyz, l0_points = x0[:, :3], x0[:, 3:]
    l1_xyz, l1_points = _sa(l0_xyz, l0_points, SA_CFGS[0], params["sa"][0])
    l2_xyz, l2_points = _sa(l1_xyz, l1_points, SA_CFGS[1], params["sa"][1])
    l3_xyz, l3_points = _sa(l2_xyz, l2_points, SA_CFGS[2], params["sa"][2])
    l4_xyz, l4_points = _sa(l3_xyz, l3_points, SA_CFGS[3], params["sa"][3])
    l3_points = _fp(l3_xyz, l4_xyz, l3_points, l4_points, params["fp"][0])
    l2_points = _fp(l2_xyz, l3_xyz, l2_points, l3_points, params["fp"][1])
    l1_points = _fp(l1_xyz, l2_xyz, l1_points, l2_points, params["fp"][2])
    l0p = _fp(l0_xyz, l1_xyz, l0_points, l1_points, params["fp"][3])
    y = _head(l0p, params["head"])
    return y[None], l0p[None]
